# baseline (speedup 1.0000x reference)
import jax
import jax.numpy as jnp
from jax.experimental import pallas as pl
from jax.experimental.pallas import tpu as pltpu

INF_ = 1e12


def _tc_body(a_ref, c_ref, r_ref, cb_ref, l_ref, x_ref, pos_ref, m_ref, k_ref):
    @pl.when(pl.program_id(1) == 0)
    def _():
        S = m_ref.shape[0]
        m = (a_ref[...] * c_ref[...]).reshape(S, S)
        m_ref[...] = m
        k_ref[...] = (INF_ * m + r_ref[...].reshape(S, 1)) + cb_ref[...].reshape(1, S)

    x = l_ref[0, 0] * m_ref[...] + k_ref[...]
    x_ref[0, 0] = x
    pos_ref[0, 0] = x > 0


def kernel(logits, attention_mask):
    B, L, S, _ = logits.shape
    af = attention_mask.astype(jnp.float32)
    rb = jnp.where((jnp.arange(S) == 0) | (jnp.arange(S) == S - 1),
                   jnp.float32(-INF_), jnp.float32(0.0))
    A = af.reshape(B, S, 1)
    C = af.reshape(B, 1, S)
    R = jnp.broadcast_to((rb - INF_).reshape(1, S, 1), (B, S, 1))
    Cb = jnp.broadcast_to(rb.reshape(1, 1, S), (B, 1, S))
    x, pos = pl.pallas_call(
        _tc_body,
        grid=(B, L),
        in_specs=[
            pl.BlockSpec((1, S, 1), lambda b, l: (b, 0, 0)),
            pl.BlockSpec((1, 1, S), lambda b, l: (b, 0, 0)),
            pl.BlockSpec((1, S, 1), lambda b, l: (b, 0, 0)),
            pl.BlockSpec((1, 1, S), lambda b, l: (b, 0, 0)),
            pl.BlockSpec((1, 1, S, S), lambda b, l: (b, l, 0, 0)),
        ],
        out_specs=[
            pl.BlockSpec((1, 1, S, S), lambda b, l: (b, l, 0, 0)),
            pl.BlockSpec((1, 1, S, S), lambda b, l: (b, l, 0, 0)),
        ],
        out_shape=[
            jax.ShapeDtypeStruct((B, L, S, S), jnp.float32),
            jax.ShapeDtypeStruct((B, L, S, S), jnp.bool_),
        ],
        scratch_shapes=[
            pltpu.VMEM((S, S), jnp.float32),
            pltpu.VMEM((S, S), jnp.float32),
        ],
    )(A, C, R, Cb, logits)
    return x, pos
